# Initial kernel scaffold; baseline (speedup 1.0000x reference)
import jax
import jax.numpy as jnp
from jax.experimental import pallas as pl


def kernel(x, Wg, bg, W1, b1, W2, b2):
    raise NotImplementedError("write your pallas kernel here")


# speedup vs baseline: 4.5993x; 4.5993x over previous
import functools
import math

import jax
import jax.numpy as jnp
from jax import lax
from jax.experimental import pallas as pl
from jax.experimental.pallas import tpu as pltpu
from jax.experimental.pallas import tpu_sc as plsc

S = 2048
D = 768
E = 16
BT = 128
NUM_SLOTS = S + E * BT
NB = NUM_SLOTS // BT
CH = 128


def _router_body(x_ref, wg_ref, bg_ref, probs_ref, slot_ref, be_ref):
    x = x_ref[...]
    logits = lax.dot_general(x, wg_ref[...],
                             (((1,), (1,)), ((), ())),
                             preferred_element_type=jnp.float32)
    logits = logits + bg_ref[...]
    m = jnp.max(logits, axis=1, keepdims=True)
    ex = jnp.exp(logits - m)
    probs = ex / jnp.sum(ex, axis=1, keepdims=True)
    probs_ref[...] = probs

    lane = lax.broadcasted_iota(jnp.int32, (S, E), 1).astype(jnp.float32)
    pm = jnp.max(probs, axis=1, keepdims=True)
    eidx = jnp.min(jnp.where(probs == pm, lane, float(E)), axis=1,
                   keepdims=True)
    onehot = (lane == eidx).astype(jnp.float32)

    r = lax.broadcasted_iota(jnp.int32, (CH, CH), 0)
    c = lax.broadcasted_iota(jnp.int32, (CH, CH), 1)
    tril = (c <= r).astype(jnp.float32)
    run = jnp.zeros((1, E), jnp.float32)
    chunks = []
    for i in range(S // CH):
        oh = onehot[i * CH:(i + 1) * CH]
        chunks.append(jnp.dot(tril, oh, preferred_element_type=jnp.float32)
                      + run)
        run = run + jnp.sum(oh, axis=0, keepdims=True)
    prefix = jnp.concatenate(chunks, axis=0)
    rank = jnp.sum(onehot * (prefix - 1.0), axis=1, keepdims=True)

    counts = run.astype(jnp.int32)
    padded = ((counts + (BT - 1)) >> 7) << 7
    r16 = lax.broadcasted_iota(jnp.int32, (E, E), 0)
    c16 = lax.broadcasted_iota(jnp.int32, (E, E), 1)
    supper = (r16 < c16).astype(jnp.float32)
    pstarts = jnp.dot(padded.astype(jnp.float32), supper,
                      preferred_element_type=jnp.float32)

    slot = jnp.sum(onehot * pstarts, axis=1, keepdims=True) + rank
    slot_ref[...] = slot.astype(jnp.int32)

    bstart = lax.broadcasted_iota(jnp.int32, (NB, 1), 0).astype(jnp.float32) * float(BT)
    owner = jnp.sum((pstarts <= bstart).astype(jnp.float32), axis=1,
                    keepdims=True) - 1.0
    be_ref[...] = owner.astype(jnp.int32)


def _router(x2, Wg, bg2):
    return pl.pallas_call(
        _router_body,
        out_shape=[
            jax.ShapeDtypeStruct((S, E), jnp.float32),
            jax.ShapeDtypeStruct((S, 1), jnp.int32),
            jax.ShapeDtypeStruct((NB, 1), jnp.int32),
        ],
    )(x2, Wg, bg2)


def _ffn_body(be_ref, x_ref, w1_ref, b1_ref, w2_ref, b2_ref, y_ref):
    x = x_ref[...]
    h = jnp.dot(x, w1_ref[0], preferred_element_type=jnp.float32)
    h = h + b1_ref[0]
    h = 0.5 * h * (1.0 + lax.erf(h * (1.0 / math.sqrt(2.0))))
    y = jnp.dot(h, w2_ref[0], preferred_element_type=jnp.float32)
    y_ref[...] = y + b2_ref[0]


def _ffn(be, xs, W1, b1, W2, b2):
    grid_spec = pltpu.PrefetchScalarGridSpec(
        num_scalar_prefetch=1,
        grid=(NB,),
        in_specs=[
            pl.BlockSpec((BT, D), lambda b, be: (b, 0)),
            pl.BlockSpec((1, D, D), lambda b, be: (be[b], 0, 0)),
            pl.BlockSpec((1, 1, D), lambda b, be: (be[b], 0, 0)),
            pl.BlockSpec((1, D, D), lambda b, be: (be[b], 0, 0)),
            pl.BlockSpec((1, 1, D), lambda b, be: (be[b], 0, 0)),
        ],
        out_specs=pl.BlockSpec((BT, D), lambda b, be: (b, 0)),
    )
    return pl.pallas_call(
        _ffn_body,
        grid_spec=grid_spec,
        out_shape=jax.ShapeDtypeStruct((NUM_SLOTS, D), jnp.float32),
    )(be, xs, W1, b1.reshape(E, 1, D), W2, b2.reshape(E, 1, D))


_SC_CORES = 2
_SC_SUBCORES = 16


@functools.cache
def _make_sc_kernels():
    nw = _SC_CORES * _SC_SUBCORES
    bw = S // nw
    mesh = plsc.VectorSubcoreMesh(core_axis_name="c", subcore_axis_name="s")
    scratch = [
        pltpu.VMEM((bw,), jnp.int32),
        pltpu.VMEM((bw, D), jnp.float32),
        pltpu.SemaphoreType.DMA,
    ]

    @functools.partial(
        pl.kernel, mesh=mesh,
        out_type=jax.ShapeDtypeStruct((NUM_SLOTS, D), jnp.float32),
        scratch_types=scratch,
    )
    def dispatch(x_hbm, slot_hbm, xs_hbm, idx_v, rows_v, sem):
        wid = lax.axis_index("s") * _SC_CORES + lax.axis_index("c")
        base = wid * bw
        pltpu.sync_copy(slot_hbm.at[pl.ds(base, bw)], idx_v)
        pltpu.sync_copy(x_hbm.at[pl.ds(base, bw)], rows_v)
        pltpu.async_copy(rows_v, xs_hbm.at[idx_v], sem).wait()

    @functools.partial(
        pl.kernel, mesh=mesh,
        out_type=jax.ShapeDtypeStruct((S, D), jnp.float32),
        scratch_types=scratch,
    )
    def combine(ys_hbm, slot_hbm, out_hbm, idx_v, rows_v, sem):
        wid = lax.axis_index("s") * _SC_CORES + lax.axis_index("c")
        base = wid * bw
        pltpu.sync_copy(slot_hbm.at[pl.ds(base, bw)], idx_v)
        pltpu.async_copy(ys_hbm.at[idx_v], rows_v, sem).wait()
        pltpu.sync_copy(rows_v, out_hbm.at[pl.ds(base, bw)])

    return dispatch, combine


def kernel(x, Wg, bg, W1, b1, W2, b2):
    _dispatch, _combine = _make_sc_kernels()
    x2 = x.reshape(S, D)
    probs, slot2, be2 = _router(x2, Wg, bg.reshape(1, E))
    slot = slot2.reshape(S)
    be = be2.reshape(NB)
    xs = _dispatch(x2, slot)
    ys = _ffn(be, xs, W1, b1, W2, b2)
    out = _combine(ys, slot)
    return out.reshape(x.shape), probs.reshape(x.shape[0], S, E)
